# baseline (speedup 1.0000x reference)
import functools

import jax
import jax.numpy as jnp
from jax import lax
from jax.experimental import pallas as pl
from jax.experimental.pallas import tpu as pltpu
from jax.experimental.pallas import tpu_sc as plsc

_N = 10000
_E = 320000
_H = 128
_DE = 16
_DIN = 2 * _H + _DE

_NC = 2
_NS = 16
_NW = _NC * _NS

_K = 5
_ECH = _E // _K
_EPW = _ECH // _NW
_CH = 80
_NIT = _EPW // _CH

_NAGG = 10240
_NPT = _NAGG // _NS
_NPB = 128
_NOB = _NPT // _NPB


def _wid():
    return lax.axis_index("c") * _NS + lax.axis_index("s")


def _gather_body(z_hbm, src_hbm, dst_hbm, zsrc_hbm, zdst_hbm,
                 idx_s, idx_d, rs0, rs1, rd0, rd1, z_sh,
                 sgs0, sgs1, sgd0, sgd1, sws0, sws1, swd0, swd1):
    wid = _wid()
    s = lax.axis_index("s")
    base = wid * _EPW
    pltpu.sync_copy(z_hbm.at[pl.ds(s * _NPT, _NPT)], z_sh.at[pl.ds(s * _NPT, _NPT)])
    pltpu.sync_copy(src_hbm.at[wid], idx_s)
    pltpu.sync_copy(dst_hbm.at[wid], idx_d)
    plsc.subcore_barrier()

    rs = (rs0, rs1)
    rd = (rd0, rd1)
    sgs = (sgs0, sgs1)
    sgd = (sgd0, sgd1)
    sws = (sws0, sws1)
    swd = (swd0, swd1)
    g = [None, None]
    w = [None, None]
    g[0] = (pltpu.async_copy(z_sh.at[idx_s.at[0]], rs[0], sgs[0]),
            pltpu.async_copy(z_sh.at[idx_d.at[0]], rd[0], sgd[0]))
    for i in range(_NIT):
        b = i & 1
        g[b][0].wait()
        g[b][1].wait()
        w[b] = (pltpu.async_copy(rs[b], zsrc_hbm.at[pl.ds(base + i * _CH, _CH)],
                                 sws[b]),
                pltpu.async_copy(rd[b], zdst_hbm.at[pl.ds(base + i * _CH, _CH)],
                                 swd[b]))
        if i + 1 < _NIT:
            if i >= 1:
                w[b ^ 1][0].wait()
                w[b ^ 1][1].wait()
            g[b ^ 1] = (
                pltpu.async_copy(z_sh.at[idx_s.at[i + 1]], rs[b ^ 1], sgs[b ^ 1]),
                pltpu.async_copy(z_sh.at[idx_d.at[i + 1]], rd[b ^ 1], sgd[b ^ 1]))
    last = (_NIT - 1) & 1
    if _NIT >= 2:
        w[last ^ 1][0].wait()
        w[last ^ 1][1].wait()
    w[last][0].wait()
    w[last][1].wait()


def _sc_gather(z, src3, dst3):
    mesh = plsc.VectorSubcoreMesh(core_axis_name="c", subcore_axis_name="s",
                                  num_cores=_NC)
    k = functools.partial(
        pl.kernel,
        out_type=(jax.ShapeDtypeStruct((_ECH, _H), jnp.float32),
                  jax.ShapeDtypeStruct((_ECH, _H), jnp.float32)),
        mesh=mesh,
        scratch_types=[
            pltpu.VMEM((_NIT, _CH), jnp.int32),
            pltpu.VMEM((_NIT, _CH), jnp.int32),
            pltpu.VMEM((_CH, _H), jnp.float32),
            pltpu.VMEM((_CH, _H), jnp.float32),
            pltpu.VMEM((_CH, _H), jnp.float32),
            pltpu.VMEM((_CH, _H), jnp.float32),
            pltpu.VMEM_SHARED((_NAGG, _H), jnp.float32),
        ] + [pltpu.SemaphoreType.DMA] * 8,
    )(_gather_body)
    return k(z, src3, dst3)


def _dense_body(zs_ref, zd_ref, ea_ref, w1t_ref, w2t_ref, w3_ref, bffw_ref,
                wf1_ref, wf2_ref, bf_ref, ws1_ref, ws2_ref, bs_ref,
                zedget_ref, msg_ref):
    zs = zs_ref[...]
    zd = zd_ref[...]
    ea_t = ea_ref[...]
    dg = functools.partial(lax.dot_general,
                           preferred_element_type=jnp.float32)
    pet = (dg(w1t_ref[...], zs, (((1,), (1,)), ((), ())))
           + dg(w2t_ref[...], zd, (((1,), (1,)), ((), ())))
           + dg(w3_ref[...], ea_t, (((0,), (0,)), ((), ())))
           + bffw_ref[...])
    zedget_ref[...] = jnp.maximum(pet, 0.0)
    dot = functools.partial(jnp.dot, preferred_element_type=jnp.float32)
    gp = dot(zd, wf1_ref[...]) + dot(zs, wf2_ref[...]) + bf_ref[...]
    cp = dot(zd, ws1_ref[...]) + dot(zs, ws2_ref[...]) + bs_ref[...]
    gate = 1.0 / (1.0 + jnp.exp(-gp))
    core = jnp.maximum(cp, 0.0) + jnp.log(1.0 + jnp.exp(-jnp.abs(cp)))
    msg_ref[...] = gate * core


def _dense_body_aliased(buf_ref, *refs):
    del buf_ref
    _dense_body(*refs)


def _tc_dense(k, zedget_buf, zsrc, zdst, ea_t,
              w1t, w2t, w3, bffw, wf1, wf2, bf, ws1, ws2, bs):
    be = 3200
    nblk = _ECH // be
    row = lambda i: (i, 0)
    colk = lambda i: (0, k * nblk + i)
    rep = lambda i: (0, 0)
    in_specs = [
        pl.BlockSpec((be, _H), row),
        pl.BlockSpec((be, _H), row),
        pl.BlockSpec((_DE, be), colk),
        pl.BlockSpec((_DIN, _H), rep),
        pl.BlockSpec((_DIN, _H), rep),
        pl.BlockSpec((_DE, _DIN), rep),
        pl.BlockSpec((_DIN, 1), rep),
        pl.BlockSpec((_H, _H), rep),
        pl.BlockSpec((_H, _H), rep),
        pl.BlockSpec((1, _H), rep),
        pl.BlockSpec((_H, _H), rep),
        pl.BlockSpec((_H, _H), rep),
        pl.BlockSpec((1, _H), rep),
    ]
    args = (zsrc, zdst, ea_t, w1t, w2t, w3, bffw, wf1, wf2, bf, ws1, ws2, bs)
    body = _dense_body
    aliases = {}
    if zedget_buf is not None:
        in_specs = [pl.BlockSpec(memory_space=pl.ANY)] + in_specs
        args = (zedget_buf,) + args
        body = _dense_body_aliased
        aliases = {0: 0}
    return pl.pallas_call(
        body,
        grid=(nblk,),
        in_specs=in_specs,
        out_specs=[
            pl.BlockSpec((_DIN, be), colk),
            pl.BlockSpec((be, _H), row),
        ],
        out_shape=[
            jax.ShapeDtypeStruct((_DIN, _E), jnp.float32),
            jax.ShapeDtypeStruct((_ECH, _H), jnp.float32),
        ],
        input_output_aliases=aliases,
    )(*args)


def _make_scatter_body(nchunks):
    def _scatter_body(*refs):
        msgs = refs[:nchunks]
        dst_hbm = refs[nchunks]
        part_hbm = refs[nchunks + 1]
        idx_d, r0b, r1b, stage, agg_sh, sem0, sem1 = refs[nchunks + 2:]
        rows = (r0b, r1b)
        sems = (sem0, sem1)
        c = lax.axis_index("c")
        s = lax.axis_index("s")
        wid = c * _NS + s

        def zr(r, carry):
            def zk(k, carry2):
                stage[r, pl.ds(k * 16, 16)] = jnp.zeros((16,), jnp.float32)
                return carry2
            return lax.fori_loop(0, _H // 16, zk, carry)

        lax.fori_loop(0, _NPB, zr, 0)

        def zcopy(j, carry):
            pltpu.sync_copy(stage, agg_sh.at[pl.ds(s * _NPT + j * _NPB, _NPB)])
            return carry

        lax.fori_loop(0, _NOB, zcopy, 0)
        for c2 in range(nchunks):
            pltpu.sync_copy(dst_hbm.at[c2, wid], idx_d.at[pl.ds(c2 * _NIT, _NIT)])
        plsc.subcore_barrier()

        total = nchunks * _NIT
        base = wid * _EPW

        def src_of(t):
            c2, i = divmod(t, _NIT)
            return msgs[c2].at[pl.ds(base + i * _CH, _CH)]

        m = [None, None]
        m[0] = pltpu.async_copy(src_of(0), rows[0], sems[0])
        for t in range(total):
            b = t & 1
            m[b].wait()
            if t + 1 < total:
                m[b ^ 1] = pltpu.async_copy(src_of(t + 1), rows[b ^ 1],
                                            sems[b ^ 1])
            pltpu.sync_copy(rows[b], agg_sh.at[idx_d.at[t]], add=True)
        plsc.subcore_barrier()

        def ocopy(j, carry):
            r0 = s * _NPT + j * _NPB
            pltpu.sync_copy(agg_sh.at[pl.ds(r0, _NPB)], stage)
            pltpu.sync_copy(stage, part_hbm.at[c, pl.ds(r0, _NPB)])
            return carry

        lax.fori_loop(0, _NOB, ocopy, 0)

    return _scatter_body


def _sc_scatter(msgs, dst4_slice):
    nchunks = len(msgs)
    mesh = plsc.VectorSubcoreMesh(core_axis_name="c", subcore_axis_name="s",
                                  num_cores=_NC)
    k = functools.partial(
        pl.kernel,
        out_type=jax.ShapeDtypeStruct((_NC, _NAGG, _H), jnp.float32),
        mesh=mesh,
        scratch_types=[
            pltpu.VMEM((nchunks * _NIT, _CH), jnp.int32),
            pltpu.VMEM((_CH, _H), jnp.float32),
            pltpu.VMEM((_CH, _H), jnp.float32),
            pltpu.VMEM((_NPB, _H), jnp.float32),
            pltpu.VMEM_SHARED((_NAGG, _H), jnp.float32),
            pltpu.SemaphoreType.DMA,
            pltpu.SemaphoreType.DMA,
        ],
    )(_make_scatter_body(nchunks))
    return k(*msgs, dst4_slice)


def _combine_body(z_ref, *refs):
    parts = refs[:-1]
    out_ref = refs[-1]
    acc = z_ref[...]
    for p in parts:
        acc = acc + p[0] + p[1]
    out_ref[...] = acc


def _tc_combine(z, parts):
    bn = 2000
    return pl.pallas_call(
        _combine_body,
        grid=(_N // bn,),
        in_specs=[pl.BlockSpec((bn, _H), lambda i: (i, 0))] +
                 [pl.BlockSpec((_NC, bn, _H), lambda i: (0, i, 0))
                  for _ in parts],
        out_specs=pl.BlockSpec((bn, _H), lambda i: (i, 0)),
        out_shape=jax.ShapeDtypeStruct((_N, _H), jnp.float32),
    )(z, *parts)


def kernel(z, edge_attr, edge_index, W_ffw, b_ffw, W_f, b_f, W_s, b_s):
    src4 = edge_index[0].reshape(_K, _NW, _NIT, _CH)
    dst4 = edge_index[1].reshape(_K, _NW, _NIT, _CH)

    w1 = W_ffw[:_H]
    w2 = W_ffw[_H:2 * _H]
    w3 = W_ffw[2 * _H:]
    wf1 = W_f[:_H]
    wf2 = W_f[_H:]
    ws1 = W_s[:_H]
    ws2 = W_s[_H:]
    bffw = b_ffw.reshape(1, _DIN)
    bf2 = b_f.reshape(1, _H)
    bs2 = b_s.reshape(1, _H)

    ea_t = edge_attr.T
    z_pad = jnp.pad(z, ((0, _NAGG - _N), (0, 0)))
    gathered = [_sc_gather(z_pad, src4[k], dst4[k]) for k in range(_K)]
    zedget = None
    msgs = []
    for k in range(_K):
        zsrc, zdst = gathered[k]
        zedget, msg_k = _tc_dense(k, zedget, zsrc, zdst, ea_t,
                                  w1.T, w2.T, w3, bffw.T,
                                  wf1, wf2, bf2, ws1, ws2, bs2)
        msgs.append(msg_k)
    z_edge = zedget.T

    parts = [_sc_scatter(msgs[0:3], dst4[0:3]),
             _sc_scatter(msgs[3:4], dst4[3:4]),
             _sc_scatter(msgs[4:5], dst4[4:5])]
    z_node = _tc_combine(z, parts)
    return (z_node, z_edge)


# speedup vs baseline: 1.0373x; 1.0373x over previous
import functools

import jax
import jax.numpy as jnp
from jax import lax
from jax.experimental import pallas as pl
from jax.experimental.pallas import tpu as pltpu
from jax.experimental.pallas import tpu_sc as plsc

_N = 10000
_E = 320000
_H = 128
_DE = 16
_DIN = 2 * _H + _DE

_NC = 2
_NS = 16
_NW = _NC * _NS

_K = 5
_ECH = _E // _K
_EPW = _ECH // _NW
_CH = 80
_NIT = _EPW // _CH

_NAGG = 10240
_NPT = _NAGG // _NS
_NPB = 128
_NOB = _NPT // _NPB


def _wid():
    return lax.axis_index("c") * _NS + lax.axis_index("s")


def _gather_body(z_hbm, src_hbm, dst_hbm, zsrc_hbm, zdst_hbm,
                 idx_s, idx_d, rs0, rs1, rd0, rd1, z_sh,
                 sgs0, sgs1, sgd0, sgd1, sws0, sws1, swd0, swd1):
    wid = _wid()
    s = lax.axis_index("s")
    base = wid * _EPW
    pltpu.sync_copy(z_hbm.at[pl.ds(s * _NPT, _NPT)], z_sh.at[pl.ds(s * _NPT, _NPT)])
    pltpu.sync_copy(src_hbm.at[wid], idx_s)
    pltpu.sync_copy(dst_hbm.at[wid], idx_d)
    plsc.subcore_barrier()

    rs = (rs0, rs1)
    rd = (rd0, rd1)
    sgs = (sgs0, sgs1)
    sgd = (sgd0, sgd1)
    sws = (sws0, sws1)
    swd = (swd0, swd1)
    g = [None, None]
    w = [None, None]
    g[0] = (pltpu.async_copy(z_sh.at[idx_s.at[0]], rs[0], sgs[0]),
            pltpu.async_copy(z_sh.at[idx_d.at[0]], rd[0], sgd[0]))
    for i in range(_NIT):
        b = i & 1
        g[b][0].wait()
        g[b][1].wait()
        w[b] = (pltpu.async_copy(rs[b], zsrc_hbm.at[pl.ds(base + i * _CH, _CH)],
                                 sws[b]),
                pltpu.async_copy(rd[b], zdst_hbm.at[pl.ds(base + i * _CH, _CH)],
                                 swd[b]))
        if i + 1 < _NIT:
            if i >= 1:
                w[b ^ 1][0].wait()
                w[b ^ 1][1].wait()
            g[b ^ 1] = (
                pltpu.async_copy(z_sh.at[idx_s.at[i + 1]], rs[b ^ 1], sgs[b ^ 1]),
                pltpu.async_copy(z_sh.at[idx_d.at[i + 1]], rd[b ^ 1], sgd[b ^ 1]))
    last = (_NIT - 1) & 1
    if _NIT >= 2:
        w[last ^ 1][0].wait()
        w[last ^ 1][1].wait()
    w[last][0].wait()
    w[last][1].wait()


def _sc_gather(z, src3, dst3):
    mesh = plsc.VectorSubcoreMesh(core_axis_name="c", subcore_axis_name="s",
                                  num_cores=_NC)
    k = functools.partial(
        pl.kernel,
        out_type=(jax.ShapeDtypeStruct((_ECH, _H), jnp.float32),
                  jax.ShapeDtypeStruct((_ECH, _H), jnp.float32)),
        mesh=mesh,
        scratch_types=[
            pltpu.VMEM((_NIT, _CH), jnp.int32),
            pltpu.VMEM((_NIT, _CH), jnp.int32),
            pltpu.VMEM((_CH, _H), jnp.float32),
            pltpu.VMEM((_CH, _H), jnp.float32),
            pltpu.VMEM((_CH, _H), jnp.float32),
            pltpu.VMEM((_CH, _H), jnp.float32),
            pltpu.VMEM_SHARED((_NAGG, _H), jnp.float32),
        ] + [pltpu.SemaphoreType.DMA] * 8,
    )(_gather_body)
    return k(z, src3, dst3)


def _dense_body(zs_ref, zd_ref, ea_ref, w1t_ref, w2t_ref, w3_ref, bffw_ref,
                wf1_ref, wf2_ref, bf_ref, ws1_ref, ws2_ref, bs_ref,
                zedget_ref, msg_ref):
    zs = zs_ref[...].astype(jnp.bfloat16)
    zd = zd_ref[...].astype(jnp.bfloat16)
    ea_t = ea_ref[...]
    dg = functools.partial(lax.dot_general,
                           preferred_element_type=jnp.float32)
    pet = (dg(w1t_ref[...], zs, (((1,), (1,)), ((), ())))
           + dg(w2t_ref[...], zd, (((1,), (1,)), ((), ())))
           + dg(w3_ref[...], ea_t, (((0,), (0,)), ((), ())))
           + bffw_ref[...])
    zedget_ref[...] = jnp.maximum(pet, 0.0)
    dot = functools.partial(jnp.dot, preferred_element_type=jnp.float32)
    gp = dot(zd, wf1_ref[...]) + dot(zs, wf2_ref[...]) + bf_ref[...]
    cp = dot(zd, ws1_ref[...]) + dot(zs, ws2_ref[...]) + bs_ref[...]
    gate = 1.0 / (1.0 + jnp.exp(-gp))
    core = jnp.maximum(cp, 0.0) + jnp.log(1.0 + jnp.exp(-jnp.abs(cp)))
    msg_ref[...] = gate * core


def _dense_body_aliased(buf_ref, *refs):
    del buf_ref
    _dense_body(*refs)


def _tc_dense(k, zedget_buf, zsrc, zdst, ea_t,
              w1t, w2t, w3, bffw, wf1, wf2, bf, ws1, ws2, bs):
    be = 3200
    nblk = _ECH // be
    row = lambda i: (i, 0)
    colk = lambda i: (0, k * nblk + i)
    rep = lambda i: (0, 0)
    in_specs = [
        pl.BlockSpec((be, _H), row),
        pl.BlockSpec((be, _H), row),
        pl.BlockSpec((_DE, be), colk),
        pl.BlockSpec((_DIN, _H), rep),
        pl.BlockSpec((_DIN, _H), rep),
        pl.BlockSpec((_DE, _DIN), rep),
        pl.BlockSpec((_DIN, 1), rep),
        pl.BlockSpec((_H, _H), rep),
        pl.BlockSpec((_H, _H), rep),
        pl.BlockSpec((1, _H), rep),
        pl.BlockSpec((_H, _H), rep),
        pl.BlockSpec((_H, _H), rep),
        pl.BlockSpec((1, _H), rep),
    ]
    args = (zsrc, zdst, ea_t, w1t, w2t, w3, bffw, wf1, wf2, bf, ws1, ws2, bs)
    body = _dense_body
    aliases = {}
    if zedget_buf is not None:
        in_specs = [pl.BlockSpec(memory_space=pl.ANY)] + in_specs
        args = (zedget_buf,) + args
        body = _dense_body_aliased
        aliases = {0: 0}
    return pl.pallas_call(
        body,
        grid=(nblk,),
        in_specs=in_specs,
        out_specs=[
            pl.BlockSpec((_DIN, be), colk),
            pl.BlockSpec((be, _H), row),
        ],
        out_shape=[
            jax.ShapeDtypeStruct((_DIN, _E), jnp.float32),
            jax.ShapeDtypeStruct((_ECH, _H), jnp.float32),
        ],
        input_output_aliases=aliases,
    )(*args)


def _make_scatter_body(nchunks):
    def _scatter_body(*refs):
        msgs = refs[:nchunks]
        dst_hbm = refs[nchunks]
        part_hbm = refs[nchunks + 1]
        idx_d, r0b, r1b, stage, agg_sh, sem0, sem1 = refs[nchunks + 2:]
        rows = (r0b, r1b)
        sems = (sem0, sem1)
        c = lax.axis_index("c")
        s = lax.axis_index("s")
        wid = c * _NS + s

        def zr(r, carry):
            def zk(k, carry2):
                stage[r, pl.ds(k * 16, 16)] = jnp.zeros((16,), jnp.float32)
                return carry2
            return lax.fori_loop(0, _H // 16, zk, carry)

        lax.fori_loop(0, _NPB, zr, 0)

        def zcopy(j, carry):
            pltpu.sync_copy(stage, agg_sh.at[pl.ds(s * _NPT + j * _NPB, _NPB)])
            return carry

        lax.fori_loop(0, _NOB, zcopy, 0)
        for c2 in range(nchunks):
            pltpu.sync_copy(dst_hbm.at[c2, wid], idx_d.at[pl.ds(c2 * _NIT, _NIT)])
        plsc.subcore_barrier()

        total = nchunks * _NIT
        base = wid * _EPW

        def src_of(t):
            c2, i = divmod(t, _NIT)
            return msgs[c2].at[pl.ds(base + i * _CH, _CH)]

        m = [None, None]
        m[0] = pltpu.async_copy(src_of(0), rows[0], sems[0])
        for t in range(total):
            b = t & 1
            m[b].wait()
            if t + 1 < total:
                m[b ^ 1] = pltpu.async_copy(src_of(t + 1), rows[b ^ 1],
                                            sems[b ^ 1])
            pltpu.sync_copy(rows[b], agg_sh.at[idx_d.at[t]], add=True)
        plsc.subcore_barrier()

        def ocopy(j, carry):
            r0 = s * _NPT + j * _NPB
            pltpu.sync_copy(agg_sh.at[pl.ds(r0, _NPB)], stage)
            pltpu.sync_copy(stage, part_hbm.at[c, pl.ds(r0, _NPB)])
            return carry

        lax.fori_loop(0, _NOB, ocopy, 0)

    return _scatter_body


def _sc_scatter(msgs, dst4_slice):
    nchunks = len(msgs)
    mesh = plsc.VectorSubcoreMesh(core_axis_name="c", subcore_axis_name="s",
                                  num_cores=_NC)
    k = functools.partial(
        pl.kernel,
        out_type=jax.ShapeDtypeStruct((_NC, _NAGG, _H), jnp.float32),
        mesh=mesh,
        scratch_types=[
            pltpu.VMEM((nchunks * _NIT, _CH), jnp.int32),
            pltpu.VMEM((_CH, _H), jnp.float32),
            pltpu.VMEM((_CH, _H), jnp.float32),
            pltpu.VMEM((_NPB, _H), jnp.float32),
            pltpu.VMEM_SHARED((_NAGG, _H), jnp.float32),
            pltpu.SemaphoreType.DMA,
            pltpu.SemaphoreType.DMA,
        ],
    )(_make_scatter_body(nchunks))
    return k(*msgs, dst4_slice)


def _combine_body(z_ref, *refs):
    parts = refs[:-1]
    out_ref = refs[-1]
    acc = z_ref[...]
    for p in parts:
        acc = acc + p[0] + p[1]
    out_ref[...] = acc


def _tc_combine(z, parts):
    bn = 2000
    return pl.pallas_call(
        _combine_body,
        grid=(_N // bn,),
        in_specs=[pl.BlockSpec((bn, _H), lambda i: (i, 0))] +
                 [pl.BlockSpec((_NC, bn, _H), lambda i: (0, i, 0))
                  for _ in parts],
        out_specs=pl.BlockSpec((bn, _H), lambda i: (i, 0)),
        out_shape=jax.ShapeDtypeStruct((_N, _H), jnp.float32),
    )(z, *parts)


def kernel(z, edge_attr, edge_index, W_ffw, b_ffw, W_f, b_f, W_s, b_s):
    src4 = edge_index[0].reshape(_K, _NW, _NIT, _CH)
    dst4 = edge_index[1].reshape(_K, _NW, _NIT, _CH)

    w1 = W_ffw[:_H]
    w2 = W_ffw[_H:2 * _H]
    w3 = W_ffw[2 * _H:]
    wf1 = W_f[:_H]
    wf2 = W_f[_H:]
    ws1 = W_s[:_H]
    ws2 = W_s[_H:]
    bffw = b_ffw.reshape(1, _DIN)
    bf2 = b_f.reshape(1, _H)
    bs2 = b_s.reshape(1, _H)

    ea_t = edge_attr.T
    bf16 = jnp.bfloat16
    z_pad = jnp.pad(z, ((0, _NAGG - _N), (0, 0)))
    gathered = [_sc_gather(z_pad, src4[k], dst4[k]) for k in range(_K)]
    zedget = None
    msgs = []
    for k in range(_K):
        zsrc, zdst = gathered[k]
        zedget, msg_k = _tc_dense(k, zedget, zsrc, zdst, ea_t,
                                  w1.T.astype(bf16), w2.T.astype(bf16), w3,
                                  bffw.T,
                                  wf1.astype(bf16), wf2.astype(bf16), bf2,
                                  ws1.astype(bf16), ws2.astype(bf16), bs2)
        msgs.append(msg_k)
    z_edge = zedget.T

    parts = [_sc_scatter(msgs[0:3], dst4[0:3]),
             _sc_scatter(msgs[3:5], dst4[3:5])]
    z_node = _tc_combine(z, parts)
    return (z_node, z_edge)


# speedup vs baseline: 1.0397x; 1.0023x over previous
import functools

import jax
import jax.numpy as jnp
from jax import lax
from jax.experimental import pallas as pl
from jax.experimental.pallas import tpu as pltpu
from jax.experimental.pallas import tpu_sc as plsc

_N = 10000
_E = 320000
_H = 128
_DE = 16
_DIN = 2 * _H + _DE

_NC = 2
_NS = 16
_NW = _NC * _NS

_K = 5
_ECH = _E // _K
_EPW = _ECH // _NW
_CH = 80
_NIT = _EPW // _CH

_NAGG = 10240
_NPT = _NAGG // _NS
_NPB = 128
_NOB = _NPT // _NPB


def _wid():
    return lax.axis_index("c") * _NS + lax.axis_index("s")


def _gather_body(z_hbm, src_hbm, dst_hbm, zsrc_hbm, zdst_hbm,
                 idx_s, idx_d, rs0, rs1, rd0, rd1, z_sh,
                 sgs0, sgs1, sgd0, sgd1, sws0, sws1, swd0, swd1):
    wid = _wid()
    s = lax.axis_index("s")
    base = wid * _EPW
    pltpu.sync_copy(z_hbm.at[pl.ds(s * _NPT, _NPT)], z_sh.at[pl.ds(s * _NPT, _NPT)])
    pltpu.sync_copy(src_hbm.at[wid], idx_s)
    pltpu.sync_copy(dst_hbm.at[wid], idx_d)
    plsc.subcore_barrier()

    rs = (rs0, rs1)
    rd = (rd0, rd1)
    sgs = (sgs0, sgs1)
    sgd = (sgd0, sgd1)
    sws = (sws0, sws1)
    swd = (swd0, swd1)
    g = [None, None]
    w = [None, None]
    g[0] = (pltpu.async_copy(z_sh.at[idx_s.at[0]], rs[0], sgs[0]),
            pltpu.async_copy(z_sh.at[idx_d.at[0]], rd[0], sgd[0]))
    for i in range(_NIT):
        b = i & 1
        g[b][0].wait()
        g[b][1].wait()
        w[b] = (pltpu.async_copy(rs[b], zsrc_hbm.at[pl.ds(base + i * _CH, _CH)],
                                 sws[b]),
                pltpu.async_copy(rd[b], zdst_hbm.at[pl.ds(base + i * _CH, _CH)],
                                 swd[b]))
        if i + 1 < _NIT:
            if i >= 1:
                w[b ^ 1][0].wait()
                w[b ^ 1][1].wait()
            g[b ^ 1] = (
                pltpu.async_copy(z_sh.at[idx_s.at[i + 1]], rs[b ^ 1], sgs[b ^ 1]),
                pltpu.async_copy(z_sh.at[idx_d.at[i + 1]], rd[b ^ 1], sgd[b ^ 1]))
    last = (_NIT - 1) & 1
    if _NIT >= 2:
        w[last ^ 1][0].wait()
        w[last ^ 1][1].wait()
    w[last][0].wait()
    w[last][1].wait()


def _sc_gather(z, src3, dst3):
    mesh = plsc.VectorSubcoreMesh(core_axis_name="c", subcore_axis_name="s",
                                  num_cores=_NC)
    k = functools.partial(
        pl.kernel,
        out_type=(jax.ShapeDtypeStruct((_ECH, _H), jnp.float32),
                  jax.ShapeDtypeStruct((_ECH, _H), jnp.float32)),
        mesh=mesh,
        scratch_types=[
            pltpu.VMEM((_NIT, _CH), jnp.int32),
            pltpu.VMEM((_NIT, _CH), jnp.int32),
            pltpu.VMEM((_CH, _H), jnp.float32),
            pltpu.VMEM((_CH, _H), jnp.float32),
            pltpu.VMEM((_CH, _H), jnp.float32),
            pltpu.VMEM((_CH, _H), jnp.float32),
            pltpu.VMEM_SHARED((_NAGG, _H), jnp.float32),
        ] + [pltpu.SemaphoreType.DMA] * 8,
    )(_gather_body)
    return k(z, src3, dst3)


def _dense_body(zs_ref, zd_ref, ea_ref, w1t_ref, w2t_ref, w3_ref, bffw_ref,
                wf1_ref, wf2_ref, bf_ref, ws1_ref, ws2_ref, bs_ref,
                zedget_ref, msg_ref):
    zs = zs_ref[...].astype(jnp.bfloat16)
    zd = zd_ref[...].astype(jnp.bfloat16)
    ea_t = ea_ref[...]
    dg = functools.partial(lax.dot_general,
                           preferred_element_type=jnp.float32)
    pet = (dg(w1t_ref[...], zs, (((1,), (1,)), ((), ())))
           + dg(w2t_ref[...], zd, (((1,), (1,)), ((), ())))
           + dg(w3_ref[...], ea_t, (((0,), (0,)), ((), ())))
           + bffw_ref[...])
    zedget_ref[...] = jnp.maximum(pet, 0.0)
    dot = functools.partial(jnp.dot, preferred_element_type=jnp.float32)
    gp = dot(zd, wf1_ref[...]) + dot(zs, wf2_ref[...]) + bf_ref[...]
    cp = dot(zd, ws1_ref[...]) + dot(zs, ws2_ref[...]) + bs_ref[...]
    gate = 1.0 / (1.0 + jnp.exp(-gp))
    core = jnp.maximum(cp, 0.0) + jnp.log(1.0 + jnp.exp(-jnp.abs(cp)))
    msg_ref[...] = gate * core


def _dense_body_aliased(buf_ref, *refs):
    del buf_ref
    _dense_body(*refs)


def _tc_dense(k, zedget_buf, zsrc, zdst, ea_t,
              w1t, w2t, w3, bffw, wf1, wf2, bf, ws1, ws2, bs):
    be = 6400
    nblk = _ECH // be
    row = lambda i: (i, 0)
    colk = lambda i: (0, k * nblk + i)
    rep = lambda i: (0, 0)
    in_specs = [
        pl.BlockSpec((be, _H), row),
        pl.BlockSpec((be, _H), row),
        pl.BlockSpec((_DE, be), colk),
        pl.BlockSpec((_DIN, _H), rep),
        pl.BlockSpec((_DIN, _H), rep),
        pl.BlockSpec((_DE, _DIN), rep),
        pl.BlockSpec((_DIN, 1), rep),
        pl.BlockSpec((_H, _H), rep),
        pl.BlockSpec((_H, _H), rep),
        pl.BlockSpec((1, _H), rep),
        pl.BlockSpec((_H, _H), rep),
        pl.BlockSpec((_H, _H), rep),
        pl.BlockSpec((1, _H), rep),
    ]
    args = (zsrc, zdst, ea_t, w1t, w2t, w3, bffw, wf1, wf2, bf, ws1, ws2, bs)
    body = _dense_body
    aliases = {}
    if zedget_buf is not None:
        in_specs = [pl.BlockSpec(memory_space=pl.ANY)] + in_specs
        args = (zedget_buf,) + args
        body = _dense_body_aliased
        aliases = {0: 0}
    return pl.pallas_call(
        body,
        grid=(nblk,),
        in_specs=in_specs,
        out_specs=[
            pl.BlockSpec((_DIN, be), colk),
            pl.BlockSpec((be, _H), row),
        ],
        out_shape=[
            jax.ShapeDtypeStruct((_DIN, _E), jnp.float32),
            jax.ShapeDtypeStruct((_ECH, _H), jnp.float32),
        ],
        input_output_aliases=aliases,
    )(*args)


def _make_scatter_body(nchunks):
    def _scatter_body(*refs):
        msgs = refs[:nchunks]
        dst_hbm = refs[nchunks]
        part_hbm = refs[nchunks + 1]
        idx_d, r0b, r1b, stage, agg_sh, sem0, sem1 = refs[nchunks + 2:]
        rows = (r0b, r1b)
        sems = (sem0, sem1)
        c = lax.axis_index("c")
        s = lax.axis_index("s")
        wid = c * _NS + s

        def zr(r, carry):
            def zk(k, carry2):
                stage[r, pl.ds(k * 16, 16)] = jnp.zeros((16,), jnp.float32)
                return carry2
            return lax.fori_loop(0, _H // 16, zk, carry)

        lax.fori_loop(0, _NPB, zr, 0)

        def zcopy(j, carry):
            pltpu.sync_copy(stage, agg_sh.at[pl.ds(s * _NPT + j * _NPB, _NPB)])
            return carry

        lax.fori_loop(0, _NOB, zcopy, 0)
        for c2 in range(nchunks):
            pltpu.sync_copy(dst_hbm.at[c2, wid], idx_d.at[pl.ds(c2 * _NIT, _NIT)])
        plsc.subcore_barrier()

        total = nchunks * _NIT
        base = wid * _EPW

        def src_of(t):
            c2, i = divmod(t, _NIT)
            return msgs[c2].at[pl.ds(base + i * _CH, _CH)]

        m = [None, None]
        m[0] = pltpu.async_copy(src_of(0), rows[0], sems[0])
        for t in range(total):
            b = t & 1
            m[b].wait()
            if t + 1 < total:
                m[b ^ 1] = pltpu.async_copy(src_of(t + 1), rows[b ^ 1],
                                            sems[b ^ 1])
            pltpu.sync_copy(rows[b], agg_sh.at[idx_d.at[t]], add=True)
        plsc.subcore_barrier()

        def ocopy(j, carry):
            r0 = s * _NPT + j * _NPB
            pltpu.sync_copy(agg_sh.at[pl.ds(r0, _NPB)], stage)
            pltpu.sync_copy(stage, part_hbm.at[c, pl.ds(r0, _NPB)])
            return carry

        lax.fori_loop(0, _NOB, ocopy, 0)

    return _scatter_body


def _sc_scatter(msgs, dst4_slice):
    nchunks = len(msgs)
    mesh = plsc.VectorSubcoreMesh(core_axis_name="c", subcore_axis_name="s",
                                  num_cores=_NC)
    k = functools.partial(
        pl.kernel,
        out_type=jax.ShapeDtypeStruct((_NC, _NAGG, _H), jnp.float32),
        mesh=mesh,
        scratch_types=[
            pltpu.VMEM((nchunks * _NIT, _CH), jnp.int32),
            pltpu.VMEM((_CH, _H), jnp.float32),
            pltpu.VMEM((_CH, _H), jnp.float32),
            pltpu.VMEM((_NPB, _H), jnp.float32),
            pltpu.VMEM_SHARED((_NAGG, _H), jnp.float32),
            pltpu.SemaphoreType.DMA,
            pltpu.SemaphoreType.DMA,
        ],
    )(_make_scatter_body(nchunks))
    return k(*msgs, dst4_slice)


def _combine_body(z_ref, *refs):
    parts = refs[:-1]
    out_ref = refs[-1]
    acc = z_ref[...]
    for p in parts:
        acc = acc + p[0] + p[1]
    out_ref[...] = acc


def _tc_combine(z, parts):
    bn = 2000
    return pl.pallas_call(
        _combine_body,
        grid=(_N // bn,),
        in_specs=[pl.BlockSpec((bn, _H), lambda i: (i, 0))] +
                 [pl.BlockSpec((_NC, bn, _H), lambda i: (0, i, 0))
                  for _ in parts],
        out_specs=pl.BlockSpec((bn, _H), lambda i: (i, 0)),
        out_shape=jax.ShapeDtypeStruct((_N, _H), jnp.float32),
    )(z, *parts)


def kernel(z, edge_attr, edge_index, W_ffw, b_ffw, W_f, b_f, W_s, b_s):
    src4 = edge_index[0].reshape(_K, _NW, _NIT, _CH)
    dst4 = edge_index[1].reshape(_K, _NW, _NIT, _CH)

    w1 = W_ffw[:_H]
    w2 = W_ffw[_H:2 * _H]
    w3 = W_ffw[2 * _H:]
    wf1 = W_f[:_H]
    wf2 = W_f[_H:]
    ws1 = W_s[:_H]
    ws2 = W_s[_H:]
    bffw = b_ffw.reshape(1, _DIN)
    bf2 = b_f.reshape(1, _H)
    bs2 = b_s.reshape(1, _H)

    ea_t = edge_attr.T
    bf16 = jnp.bfloat16
    z_pad = jnp.pad(z, ((0, _NAGG - _N), (0, 0)))
    gathered = [_sc_gather(z_pad, src4[k], dst4[k]) for k in range(_K)]
    zedget = None
    msgs = []
    for k in range(_K):
        zsrc, zdst = gathered[k]
        zedget, msg_k = _tc_dense(k, zedget, zsrc, zdst, ea_t,
                                  w1.T.astype(bf16), w2.T.astype(bf16), w3,
                                  bffw.T,
                                  wf1.astype(bf16), wf2.astype(bf16), bf2,
                                  ws1.astype(bf16), ws2.astype(bf16), bs2)
        msgs.append(msg_k)
    z_edge = zedget.T

    parts = [_sc_scatter(msgs[0:3], dst4[0:3]),
             _sc_scatter(msgs[3:5], dst4[3:5])]
    z_node = _tc_combine(z, parts)
    return (z_node, z_edge)
